# baseline (speedup 1.0000x reference)
import functools

import jax
import jax.numpy as jnp
from jax import lax
from jax.experimental import pallas as pl
from jax.experimental.pallas import tpu as pltpu
from jax.experimental.pallas import tpu_sc as plsc

_B, _N, _C, _NHEAD, _K = 4, 8192, 768, 1, 1024
_M = _B * _N
_BLK = 512
_CH = 256
_INTERPRET = False


def _erf(x):
    a = jnp.abs(x)
    t = 1.0 / (1.0 + 0.3275911 * a)
    y = 1.0 - (((((1.061405429 * t - 1.453152027) * t) + 1.421413741) * t
                - 0.284496736) * t + 0.254829592) * t * jnp.exp(-a * a)
    return jnp.sign(x) * y


def _gelu(x):
    return 0.5 * x * (1.0 + _erf(x * 0.7071067811865476))


def _scores_body(x_ref, w1_ref, b1_ref, w2_ref, b2_ref, g_ref, be_ref,
                 wt_ref, bt_ref, out_ref):
    xb = x_ref[...]
    h = xb @ w1_ref[...] + b1_ref[...]
    h = _gelu(h)
    h = h @ w2_ref[...] + b2_ref[...]
    mu = jnp.mean(h, axis=-1, keepdims=True)
    var = jnp.mean((h - mu) ** 2, axis=-1, keepdims=True)
    hn = (h - mu) / jnp.sqrt(var + 1e-5) * g_ref[...] + be_ref[...]
    a = xb + hn
    logit = a @ wt_ref[...] + bt_ref[...]
    out_ref[...] = jax.nn.sigmoid(logit)


def _scores(x2d, W1, b1, W2, b2, gamma, beta, Wt, bt):
    full = lambda r, c: pl.BlockSpec((r, c), lambda i: (0, 0))
    return pl.pallas_call(
        _scores_body,
        grid=(_M // _BLK,),
        in_specs=[
            pl.BlockSpec((_BLK, _C), lambda i: (i, 0)),
            full(_C, _C), full(1, _C), full(_C, _C), full(1, _C),
            full(1, _C), full(1, _C), full(_C, _NHEAD), full(1, _NHEAD),
        ],
        out_specs=pl.BlockSpec((_BLK, _NHEAD), lambda i: (i, 0)),
        out_shape=jax.ShapeDtypeStruct((_M, _NHEAD), jnp.float32),
        interpret=_INTERPRET,
    )(x2d, W1, b1.reshape(1, _C), W2, b2.reshape(1, _C),
      gamma.reshape(1, _C), beta.reshape(1, _C), Wt, bt.reshape(1, _NHEAD))


def _topk_body(keys_row_ref, keys_col_ref, mult_ref, idx_ref):
    mult = mult_ref[0, 0]
    keys_row = keys_row_ref[0] * mult
    jcols = lax.broadcasted_iota(jnp.int32, (1, _N), 1)
    r_iota = lax.broadcasted_iota(jnp.int32, (1, _K), 1).astype(jnp.float32)

    ones_col = jnp.ones((_N, 1), jnp.float32)

    def chunk(c, acc):
        ki = keys_col_ref[pl.ds(c * _CH, _CH), :] * mult
        gi = lax.broadcasted_iota(jnp.int32, (_CH, 1), 0) + c * _CH
        gt = (keys_row > ki).astype(jnp.float32)
        eq = ((keys_row == ki) & (jcols < gi)).astype(jnp.float32)
        rank = lax.dot_general(gt + eq, ones_col,
                               (((1,), (0,)), ((), ())))
        onehot = (rank == r_iota).astype(jnp.float32)
        gi_row = (lax.broadcasted_iota(jnp.int32, (1, _CH), 1)
                  + c * _CH).astype(jnp.float32)
        return acc + lax.dot_general(gi_row, onehot, (((1,), (0,)), ((), ())),
                                     precision=lax.Precision.HIGHEST)

    acc = lax.fori_loop(0, _N // _CH, chunk, jnp.zeros((1, _K), jnp.float32))
    idx_ref[0] = acc.astype(jnp.int32)


def _topk_idx(scores, mult):
    keys_row = scores.reshape(_B, 1, _N)
    return pl.pallas_call(
        _topk_body,
        grid=(_B,),
        in_specs=[
            pl.BlockSpec((1, 1, _N), lambda b: (b, 0, 0)),
            pl.BlockSpec((_N, 1), lambda b: (b, 0)),
            pl.BlockSpec((1, 1), lambda b: (0, 0)),
        ],
        out_specs=pl.BlockSpec((1, 1, _K), lambda b: (b, 0, 0)),
        out_shape=jax.ShapeDtypeStruct((_B, 1, _K), jnp.int32),
        interpret=_INTERPRET,
    )(keys_row, scores, mult.reshape(1, 1)).reshape(_B, _K)


def _make_sc_gather():
    info = plsc.get_sparse_core_info()
    nw = info.num_cores * info.num_subcores
    bt = _B * _K
    bpw = bt // nw
    mesh = plsc.VectorSubcoreMesh(core_axis_name="c", subcore_axis_name="s")

    @functools.partial(
        pl.kernel, mesh=mesh,
        out_type=jax.ShapeDtypeStruct((bt, _C), jnp.float32),
        compiler_params=pltpu.CompilerParams(use_tc_tiling_on_sc=True),
        scratch_types=[
            pltpu.VMEM((bpw,), jnp.int32),
            pltpu.VMEM((bpw, _C), jnp.float32),
            pltpu.SemaphoreType.DMA,
        ],
    )
    def k(table_hbm, idx_hbm, out_hbm, idx_v, rows_v, sem):
        wid = lax.axis_index("s") * info.num_cores + lax.axis_index("c")
        base = wid * bpw
        pltpu.sync_copy(idx_hbm.at[pl.ds(base, bpw)], idx_v)
        pltpu.async_copy(table_hbm.at[idx_v], rows_v, sem).wait()
        pltpu.sync_copy(rows_v, out_hbm.at[pl.ds(base, bpw)])

    return k


def kernel(x, W1, b1, W2, b2, gamma, beta, Wt, bt, adv, epoch):
    x2d = x.reshape(_M, _C)
    h = jax.nn.gelu(x @ W1 + b1, approximate=False) @ W2 + b2
    mu = jnp.mean(h, axis=-1, keepdims=True)
    var = jnp.mean((h - mu) ** 2, axis=-1, keepdims=True)
    ln = (h - mu) / jnp.sqrt(var + 1e-5) * gamma + beta
    scores = jax.nn.sigmoid((x + ln) @ Wt + bt).reshape(_M, 1)
    mult = jnp.where(adv, -1.0, 1.0).astype(jnp.float32)
    idx_local = _topk_idx(scores, mult)
    idx_flat = (idx_local
                + (jnp.arange(_B, dtype=jnp.int32) * _N)[:, None]).reshape(-1)
    q_rows = _make_sc_gather()(x2d, idx_flat)
    q_top = q_rows.reshape(_B, _K, _NHEAD * _C)
    indices_top = idx_local.reshape(_B, _NHEAD, _K, 1)
    return (q_top, indices_top)


# speedup vs baseline: 1.1146x; 1.1146x over previous
import functools

import jax
import jax.numpy as jnp
from jax import lax
from jax.experimental import pallas as pl
from jax.experimental.pallas import tpu as pltpu
from jax.experimental.pallas import tpu_sc as plsc

_B, _N, _C, _NHEAD, _K = 4, 8192, 768, 1, 1024
_M = _B * _N
_BLK = 512
_CH = 512
_INTERPRET = False


def _erf(x):
    a = jnp.abs(x)
    t = 1.0 / (1.0 + 0.3275911 * a)
    y = 1.0 - (((((1.061405429 * t - 1.453152027) * t) + 1.421413741) * t
                - 0.284496736) * t + 0.254829592) * t * jnp.exp(-a * a)
    return jnp.sign(x) * y


def _gelu(x):
    return 0.5 * x * (1.0 + _erf(x * 0.7071067811865476))


def _scores_body(x_ref, w1_ref, b1_ref, w2_ref, b2_ref, g_ref, be_ref,
                 wt_ref, bt_ref, out_ref):
    xb = x_ref[...]
    h = xb @ w1_ref[...] + b1_ref[...]
    h = _gelu(h)
    h = h @ w2_ref[...] + b2_ref[...]
    mu = jnp.mean(h, axis=-1, keepdims=True)
    var = jnp.mean((h - mu) ** 2, axis=-1, keepdims=True)
    hn = (h - mu) / jnp.sqrt(var + 1e-5) * g_ref[...] + be_ref[...]
    a = xb + hn
    logit = a @ wt_ref[...] + bt_ref[...]
    out_ref[...] = jax.nn.sigmoid(logit)


def _scores(x2d, W1, b1, W2, b2, gamma, beta, Wt, bt):
    full = lambda r, c: pl.BlockSpec((r, c), lambda i: (0, 0))
    return pl.pallas_call(
        _scores_body,
        grid=(_M // _BLK,),
        in_specs=[
            pl.BlockSpec((_BLK, _C), lambda i: (i, 0)),
            full(_C, _C), full(1, _C), full(_C, _C), full(1, _C),
            full(1, _C), full(1, _C), full(_C, _NHEAD), full(1, _NHEAD),
        ],
        out_specs=pl.BlockSpec((_BLK, _NHEAD), lambda i: (i, 0)),
        out_shape=jax.ShapeDtypeStruct((_M, _NHEAD), jnp.float32),
        interpret=_INTERPRET,
    )(x2d, W1, b1.reshape(1, _C), W2, b2.reshape(1, _C),
      gamma.reshape(1, _C), beta.reshape(1, _C), Wt, bt.reshape(1, _NHEAD))


def _topk_body(keys_row_ref, keys_col_ref, mult_ref, idx_ref):
    mult = mult_ref[0, 0]
    keys_row = keys_row_ref[0] * mult
    jcols = lax.broadcasted_iota(jnp.int32, (1, _N), 1)
    r_iota = lax.broadcasted_iota(jnp.int32, (1, _K), 1).astype(jnp.float32)

    def chunk(c, acc):
        ki = keys_col_ref[pl.ds(c * _CH, _CH), :] * mult
        gi = lax.broadcasted_iota(jnp.int32, (_CH, 1), 0) + c * _CH
        gt = (keys_row > ki).astype(jnp.float32)
        eq = ((keys_row == ki) & (jcols < gi)).astype(jnp.float32)
        rank = jnp.sum(gt + eq, axis=1, keepdims=True)
        onehot = (rank == r_iota).astype(jnp.float32)
        gif = gi.astype(jnp.float32)
        return acc + jnp.sum(onehot * gif, axis=0, keepdims=True)

    acc = lax.fori_loop(0, _N // _CH, chunk, jnp.zeros((1, _K), jnp.float32))
    idx_ref[0] = acc.astype(jnp.int32)


def _topk_idx(scores, mult):
    keys_row = scores.reshape(_B, 1, _N)
    return pl.pallas_call(
        _topk_body,
        grid=(_B,),
        in_specs=[
            pl.BlockSpec((1, 1, _N), lambda b: (b, 0, 0)),
            pl.BlockSpec((_N, 1), lambda b: (b, 0)),
            pl.BlockSpec((1, 1), lambda b: (0, 0)),
        ],
        out_specs=pl.BlockSpec((1, 1, _K), lambda b: (b, 0, 0)),
        out_shape=jax.ShapeDtypeStruct((_B, 1, _K), jnp.int32),
        interpret=_INTERPRET,
    )(keys_row, scores, mult.reshape(1, 1)).reshape(_B, _K)


def _make_sc_gather():
    info = plsc.get_sparse_core_info()
    nw = info.num_cores * info.num_subcores
    bt = _B * _K
    bpw = bt // nw
    mesh = plsc.VectorSubcoreMesh(core_axis_name="c", subcore_axis_name="s")

    @functools.partial(
        pl.kernel, mesh=mesh,
        out_type=jax.ShapeDtypeStruct((bt, _C), jnp.float32),
        compiler_params=pltpu.CompilerParams(use_tc_tiling_on_sc=True),
        scratch_types=[
            pltpu.VMEM((bpw,), jnp.int32),
            pltpu.VMEM((bpw, _C), jnp.float32),
            pltpu.SemaphoreType.DMA,
        ],
    )
    def k(table_hbm, idx_hbm, out_hbm, idx_v, rows_v, sem):
        wid = lax.axis_index("s") * info.num_cores + lax.axis_index("c")
        base = wid * bpw
        pltpu.sync_copy(idx_hbm.at[pl.ds(base, bpw)], idx_v)
        pltpu.async_copy(table_hbm.at[idx_v], rows_v, sem).wait()
        pltpu.sync_copy(rows_v, out_hbm.at[pl.ds(base, bpw)])

    return k


def kernel(x, W1, b1, W2, b2, gamma, beta, Wt, bt, adv, epoch):
    x2d = x.reshape(_M, _C)
    h = jax.nn.gelu(x @ W1 + b1, approximate=False) @ W2 + b2
    mu = jnp.mean(h, axis=-1, keepdims=True)
    var = jnp.mean((h - mu) ** 2, axis=-1, keepdims=True)
    ln = (h - mu) / jnp.sqrt(var + 1e-5) * gamma + beta
    scores = jax.nn.sigmoid((x + ln) @ Wt + bt).reshape(_M, 1)
    mult = jnp.where(adv, -1.0, 1.0).astype(jnp.float32)
    idx_local = _topk_idx(scores, mult)
    idx_flat = (idx_local
                + (jnp.arange(_B, dtype=jnp.int32) * _N)[:, None]).reshape(-1)
    q_rows = _make_sc_gather()(x2d, idx_flat)
    q_top = q_rows.reshape(_B, _K, _NHEAD * _C)
    indices_top = idx_local.reshape(_B, _NHEAD, _K, 1)
    return (q_top, indices_top)


# speedup vs baseline: 1.1160x; 1.0013x over previous
import functools

import jax
import jax.numpy as jnp
from jax import lax
from jax.experimental import pallas as pl
from jax.experimental.pallas import tpu as pltpu
from jax.experimental.pallas import tpu_sc as plsc

_B, _N, _C, _NHEAD, _K = 4, 8192, 768, 1, 1024
_M = _B * _N
_CH = 512


def _topk_body(keys_row_ref, keys_col_ref, mult_ref, idx_ref):
    mult = mult_ref[0, 0]
    keys_row = keys_row_ref[0] * mult
    jcols = lax.broadcasted_iota(jnp.int32, (1, _N), 1)
    r_iota = lax.broadcasted_iota(jnp.int32, (1, _K), 1).astype(jnp.float32)

    def chunk(c, acc):
        ki = keys_col_ref[pl.ds(c * _CH, _CH), :] * mult
        gi = lax.broadcasted_iota(jnp.int32, (_CH, 1), 0) + c * _CH
        gt = (keys_row > ki).astype(jnp.float32)
        eq = ((keys_row == ki) & (jcols < gi)).astype(jnp.float32)
        rank = jnp.sum(gt + eq, axis=1, keepdims=True)
        onehot = (rank == r_iota).astype(jnp.float32)
        gif = gi.astype(jnp.float32)
        return acc + jnp.sum(onehot * gif, axis=0, keepdims=True)

    acc = lax.fori_loop(0, _N // _CH, chunk, jnp.zeros((1, _K), jnp.float32))
    idx_ref[0] = acc.astype(jnp.int32)


def _topk_idx(scores, mult):
    keys_row = scores.reshape(_B, 1, _N)
    return pl.pallas_call(
        _topk_body,
        grid=(_B,),
        in_specs=[
            pl.BlockSpec((1, 1, _N), lambda b: (b, 0, 0)),
            pl.BlockSpec((_N, 1), lambda b: (b, 0)),
            pl.BlockSpec((1, 1), lambda b: (0, 0)),
        ],
        out_specs=pl.BlockSpec((1, 1, _K), lambda b: (b, 0, 0)),
        out_shape=jax.ShapeDtypeStruct((_B, 1, _K), jnp.int32),
    )(keys_row, scores, mult.reshape(1, 1)).reshape(_B, _K)


def _make_sc_gather():
    info = plsc.get_sparse_core_info()
    nw = info.num_cores * info.num_subcores
    bt = _B * _K
    bpw = bt // nw
    mesh = plsc.VectorSubcoreMesh(core_axis_name="c", subcore_axis_name="s")

    @functools.partial(
        pl.kernel, mesh=mesh,
        out_type=jax.ShapeDtypeStruct((bt, _C), jnp.float32),
        compiler_params=pltpu.CompilerParams(use_tc_tiling_on_sc=True),
        scratch_types=[
            pltpu.VMEM((bpw,), jnp.int32),
            pltpu.VMEM((bpw, _C), jnp.float32),
            pltpu.SemaphoreType.DMA,
        ],
    )
    def k(table_hbm, idx_hbm, out_hbm, idx_v, rows_v, sem):
        wid = lax.axis_index("s") * info.num_cores + lax.axis_index("c")
        base = wid * bpw
        pltpu.sync_copy(idx_hbm.at[pl.ds(base, bpw)], idx_v)
        pltpu.async_copy(table_hbm.at[idx_v], rows_v, sem).wait()
        pltpu.sync_copy(rows_v, out_hbm.at[pl.ds(base, bpw)])

    return k


def kernel(x, W1, b1, W2, b2, gamma, beta, Wt, bt, adv, epoch):
    x2d = x.reshape(_M, _C)
    h = jax.nn.gelu(x @ W1 + b1, approximate=False) @ W2 + b2
    mu = jnp.mean(h, axis=-1, keepdims=True)
    var = jnp.mean((h - mu) ** 2, axis=-1, keepdims=True)
    ln = (h - mu) / jnp.sqrt(var + 1e-5) * gamma + beta
    scores = jax.nn.sigmoid((x + ln) @ Wt + bt).reshape(_M, 1)
    mult = jnp.where(adv, -1.0, 1.0).astype(jnp.float32)
    idx_local = _topk_idx(scores, mult)
    idx_flat = (idx_local
                + (jnp.arange(_B, dtype=jnp.int32) * _N)[:, None]).reshape(-1)
    q_rows = _make_sc_gather()(x2d, idx_flat)
    q_top = q_rows.reshape(_B, _K, _NHEAD * _C)
    indices_top = idx_local.reshape(_B, _NHEAD, _K, 1)
    return (q_top, indices_top)
